# baseline (speedup 1.0000x reference)
import functools

import jax
import jax.numpy as jnp
from jax import lax
from jax.experimental import pallas as pl
from jax.experimental.pallas import tpu as pltpu
from jax.experimental.pallas import tpu_sc as plsc

_B, _N, _L, _K = 4, 8, 2048, 1024
_NC = 1


def _sc_gather_body(pe_hbm, rows_hbm, idx_v, rows_v, sem):
    wid = lax.axis_index("s") * _NC + lax.axis_index("c")

    @pl.when(wid == 0)
    def _():
        i = lax.iota(jnp.int32, 16)
        idx_v[...] = jnp.where(i == 0, 0, 1)
        pltpu.async_copy(pe_hbm.at[idx_v], rows_v, sem).wait()
        pltpu.sync_copy(rows_v, rows_hbm.at[:, 0])


_sc_gather = functools.partial(
    pl.kernel,
    mesh=plsc.VectorSubcoreMesh(core_axis_name="c", subcore_axis_name="s",
                                num_cores=_NC),
    out_type=jax.ShapeDtypeStruct((16, 1, _K), jnp.float32),
    scratch_types=[
        pltpu.VMEM((16,), jnp.int32),
        pltpu.VMEM((16, _K), jnp.float32),
        pltpu.SemaphoreType.DMA,
    ],
)(_sc_gather_body)


def _tc_add_body(x_ref, row_ref, o_ref):
    o_ref[...] = x_ref[...] + row_ref[...][None]


def _tc_fixup_body(x_ref, rows_ref, _prev_ref, o_ref):
    o_ref[...] = x_ref[...] + rows_ref[...][None]


def kernel(x, pe):
    pe3 = pe.reshape(2, 1, _K)
    bulk = pl.pallas_call(
        _tc_add_body,
        grid=(_B, _N - 1),
        in_specs=[
            pl.BlockSpec((1, 1, _L, _K), lambda b, n: (b, n + 1, 0, 0)),
            pl.BlockSpec((1, 1, _K), lambda b, n: (1, 0, 0)),
        ],
        out_specs=pl.BlockSpec((1, 1, _L, _K), lambda b, n: (b, n + 1, 0, 0)),
        out_shape=jax.ShapeDtypeStruct((_B, _N, _L, _K), x.dtype),
    )(x, pe3)
    rows = _sc_gather(pe)
    return pl.pallas_call(
        _tc_fixup_body,
        grid=(_B,),
        in_specs=[
            pl.BlockSpec((1, 1, _L, _K), lambda b: (b, 0, 0, 0)),
            pl.BlockSpec((1, 1, _K), lambda b: (0, 0, 0)),
            pl.BlockSpec(memory_space=pl.ANY),
        ],
        out_specs=pl.BlockSpec((1, 1, _L, _K), lambda b: (b, 0, 0, 0)),
        out_shape=jax.ShapeDtypeStruct((_B, _N, _L, _K), x.dtype),
        input_output_aliases={2: 0},
    )(x, rows, bulk)


# speedup vs baseline: 1.0010x; 1.0010x over previous
import functools

import jax
import jax.numpy as jnp
from jax import lax
from jax.experimental import pallas as pl
from jax.experimental.pallas import tpu as pltpu
from jax.experimental.pallas import tpu_sc as plsc

_B, _N, _L, _K = 4, 8, 2048, 1024
_NC = 1


def _sc_gather_body(pe_hbm, rows_hbm, idx_v, rows_v, sem):
    wid = lax.axis_index("s") * _NC + lax.axis_index("c")

    @pl.when(wid == 0)
    def _():
        i = lax.iota(jnp.int32, 16)
        idx_v[...] = jnp.where(i == 0, 0, 1)
        pltpu.async_copy(pe_hbm.at[idx_v], rows_v, sem).wait()
        pltpu.sync_copy(rows_v, rows_hbm.at[:, 0])


_sc_gather = functools.partial(
    pl.kernel,
    mesh=plsc.VectorSubcoreMesh(core_axis_name="c", subcore_axis_name="s",
                                num_cores=_NC),
    out_type=jax.ShapeDtypeStruct((16, 1, _K), jnp.float32),
    scratch_types=[
        pltpu.VMEM((16,), jnp.int32),
        pltpu.VMEM((16, _K), jnp.float32),
        pltpu.SemaphoreType.DMA,
    ],
)(_sc_gather_body)


def _tc_add_body(x_ref, row_ref, o_ref):
    o_ref[...] = x_ref[...] + row_ref[...][None]


def _tc_fixup_body(x_ref, rows_ref, _prev_ref, o_ref):
    o_ref[...] = x_ref[...] + rows_ref[...][None]


def kernel(x, pe):
    rows = _sc_gather(pe)
    pe3 = pe.reshape(2, 1, _K)
    bulk = pl.pallas_call(
        _tc_add_body,
        grid=(_B, _N - 1),
        in_specs=[
            pl.BlockSpec((1, 1, _L, _K), lambda b, n: (b, n + 1, 0, 0)),
            pl.BlockSpec((1, 1, _K), lambda b, n: (1, 0, 0)),
        ],
        out_specs=pl.BlockSpec((1, 1, _L, _K), lambda b, n: (b, n + 1, 0, 0)),
        out_shape=jax.ShapeDtypeStruct((_B, _N, _L, _K), x.dtype),
    )(x, pe3)
    return pl.pallas_call(
        _tc_fixup_body,
        grid=(_B,),
        in_specs=[
            pl.BlockSpec((1, 1, _L, _K), lambda b: (b, 0, 0, 0)),
            pl.BlockSpec((1, 1, _K), lambda b: (0, 0, 0)),
            pl.BlockSpec(memory_space=pl.ANY),
        ],
        out_specs=pl.BlockSpec((1, 1, _L, _K), lambda b: (b, 0, 0, 0)),
        out_shape=jax.ShapeDtypeStruct((_B, _N, _L, _K), x.dtype),
        input_output_aliases={2: 0},
    )(x, rows, bulk)
